# baseline (speedup 1.0000x reference)
import functools

import jax
import jax.numpy as jnp
from jax import lax
from jax.experimental import pallas as pl
from jax.experimental.pallas import tpu as pltpu
from jax.experimental.pallas import tpu_sc as plsc

_N = 10000
_NP = 10240
_E = 160000
_EP = 163840
_NC = 2
_NS = 16
_EPW = _EP // (_NC * _NS)
_C = 128
_RPT = _NP // _NS
_BR = 512


_NB = 2
_CPW = _EPW // _C
_NR = _CPW // _NB


def _mega_depth(w):
    if w <= 16:
        return _EPW
    if w == 32:
        return _EPW // 2
    if w == 64:
        return _EPW // 5
    return None


def _sc_aggregate_mega(table, srcf, dstf, w):
    D = _mega_depth(w)
    nd = _EPW // D
    mesh = plsc.VectorSubcoreMesh(core_axis_name="c", subcore_axis_name="s")

    @functools.partial(
        pl.kernel,
        out_type=jax.ShapeDtypeStruct((_NC, _NP, w), jnp.float32),
        mesh=mesh,
        scratch_types=[
            pltpu.VMEM((_EPW,), jnp.int32),
            pltpu.VMEM((_EPW,), jnp.int32),
            pltpu.VMEM((D, w), jnp.float32),
            pltpu.VMEM_SHARED((_NP, w), jnp.float32),
            pltpu.SemaphoreType.DMA,
            pltpu.SemaphoreType.DMA,
        ],
        compiler_params=pltpu.CompilerParams(use_tc_tiling_on_sc=False),
    )
    def agg(table_h, src_h, dst_h, zero_h, out_h, sidx, didx, rows, acc, g, s):
        cid = lax.axis_index("c")
        sid = lax.axis_index("s")
        wid = cid * _NS + sid

        ibase = pl.multiple_of(wid * _EPW, 8)
        pltpu.sync_copy(src_h.at[pl.ds(ibase, _EPW)], sidx)
        pltpu.sync_copy(dst_h.at[pl.ds(ibase, _EPW)], didx)

        roff = pl.multiple_of(sid * _RPT, 8)
        pltpu.sync_copy(zero_h, rows.at[pl.ds(0, _RPT)])
        pltpu.sync_copy(rows.at[pl.ds(0, _RPT)], acc.at[pl.ds(roff, _RPT)])
        plsc.subcore_barrier()

        if nd == 1:
            pltpu.async_copy(table_h.at[sidx], rows, g).wait()
            pltpu.async_copy(rows, acc.at[didx], s, add=True).wait()
        else:
            for d in range(nd):
                sl = pl.ds(d * D, D)
                pltpu.async_copy(table_h.at[sidx.at[sl]], rows, g).wait()
                pltpu.async_copy(rows, acc.at[didx.at[sl]], s, add=True).wait()
        plsc.subcore_barrier()

        pltpu.sync_copy(acc.at[pl.ds(roff, _RPT)], rows.at[pl.ds(0, _RPT)])
        pltpu.sync_copy(rows.at[pl.ds(0, _RPT)], out_h.at[cid, pl.ds(roff, _RPT)])

    return agg(table, srcf, dstf, jnp.zeros((_RPT, w), jnp.float32))


_CW = 64
_NCH = _EPW // _CW
_NBR = 4
_NRR = _NCH // _NBR


def _sc_aggregate(table, srcf, dstf, w):
    mesh = plsc.VectorSubcoreMesh(core_axis_name="c", subcore_axis_name="s")

    @functools.partial(
        pl.kernel,
        out_type=jax.ShapeDtypeStruct((_NC, _NP, w), jnp.float32),
        mesh=mesh,
        scratch_types=[
            pltpu.VMEM((_EPW,), jnp.int32),
            pltpu.VMEM((_EPW,), jnp.int32),
            *([pltpu.VMEM((_CW, w), jnp.float32)] * _NBR),
            pltpu.VMEM_SHARED((_NP, w), jnp.float32),
            *([pltpu.SemaphoreType.DMA] * (2 * _NBR)),
        ],
        compiler_params=pltpu.CompilerParams(use_tc_tiling_on_sc=False),
    )
    def agg(table_h, src_h, dst_h, zero_h, out_h,
            sidx, didx, r0, r1, r2, r3, acc, g0, g1, g2, g3, s0, s1, s2, s3):
        rows = [r0, r1, r2, r3]
        gsem = [g0, g1, g2, g3]
        ssem = [s0, s1, s2, s3]
        cid = lax.axis_index("c")
        sid = lax.axis_index("s")
        wid = cid * _NS + sid

        ibase = pl.multiple_of(wid * _EPW, 8)
        pltpu.sync_copy(src_h.at[pl.ds(ibase, _EPW)], sidx)
        pltpu.sync_copy(dst_h.at[pl.ds(ibase, _EPW)], didx)

        pltpu.sync_copy(zero_h, rows[0])
        for k in range(_RPT // _CW):
            off = pl.multiple_of(sid * _RPT + k * _CW, 8)
            pltpu.sync_copy(rows[0], acc.at[pl.ds(off, _CW)])
        for b in range(_NBR):
            pltpu.async_copy(table_h.at[sidx.at[pl.ds(b * _CW, _CW)]],
                             rows[b], gsem[b])
        plsc.subcore_barrier()

        def rnd(r, carry):
            for b in range(_NBR):
                k = r * _NBR + b
                sl = pl.ds(k * _CW, _CW)
                pltpu.make_async_copy(table_h.at[sidx.at[sl]], rows[b],
                                      gsem[b]).wait()
                pltpu.async_copy(rows[b], acc.at[didx.at[sl]], ssem[b],
                                 add=True)
            for b in range(_NBR):
                k = (r + 1) * _NBR + b
                psl = pl.ds((k - _NBR) * _CW, _CW)
                pltpu.make_async_copy(rows[b], acc.at[didx.at[psl]],
                                      ssem[b]).wait()
                pltpu.async_copy(table_h.at[sidx.at[pl.ds(k * _CW, _CW)]],
                                 rows[b], gsem[b])
            return carry

        lax.fori_loop(0, _NRR - 1, rnd, 0)
        for b in range(_NBR):
            k = (_NRR - 1) * _NBR + b
            sl = pl.ds(k * _CW, _CW)
            pltpu.make_async_copy(table_h.at[sidx.at[sl]], rows[b],
                                  gsem[b]).wait()
            pltpu.async_copy(rows[b], acc.at[didx.at[sl]], ssem[b], add=True)
        for b in range(_NBR):
            k = (_NRR - 1) * _NBR + b
            pltpu.make_async_copy(rows[b], acc.at[didx.at[pl.ds(k * _CW, _CW)]],
                                  ssem[b]).wait()
        plsc.subcore_barrier()

        nwb = _RPT // _CW
        for k in range(nwb):
            off = pl.multiple_of(sid * _RPT + k * _CW, 8)
            b = k % _NBR
            if k >= _NBR:
                poff = pl.multiple_of(sid * _RPT + (k - _NBR) * _CW, 8)
                pltpu.make_async_copy(rows[b], out_h.at[cid, pl.ds(poff, _CW)],
                                      ssem[b]).wait()
            pltpu.async_copy(acc.at[pl.ds(off, _CW)], rows[b], gsem[b]).wait()
            pltpu.async_copy(rows[b], out_h.at[cid, pl.ds(off, _CW)], ssem[b])
        for k in range(nwb - _NBR, nwb):
            off = pl.multiple_of(sid * _RPT + k * _CW, 8)
            pltpu.make_async_copy(rows[k % _NBR], out_h.at[cid, pl.ds(off, _CW)],
                                  ssem[k % _NBR]).wait()

    return agg(table, srcf, dstf, jnp.zeros((_CW, w), jnp.float32))


def _rows_call(fn, row_ins, whole_ins, out_widths):
    nb = _NP // _BR
    nri, nwi = len(row_ins), len(whole_ins)
    in_specs = (
        [pl.BlockSpec((_BR, a.shape[1]), lambda i: (i, 0)) for a in row_ins]
        + [pl.BlockSpec(a.shape, lambda i, _nd=a.ndim: (0,) * _nd) for a in whole_ins]
    )
    out_specs = [pl.BlockSpec((_BR, w), lambda i: (i, 0)) for w in out_widths]
    out_shape = [jax.ShapeDtypeStruct((_NP, w), jnp.float32) for w in out_widths]

    def body(*refs):
        ins = [r[...] for r in refs[: nri + nwi]]
        outs = fn(*ins)
        if not isinstance(outs, (tuple, list)):
            outs = (outs,)
        for r, o in zip(refs[nri + nwi:], outs):
            r[...] = o

    return pl.pallas_call(
        body,
        grid=(nb,),
        in_specs=in_specs,
        out_specs=out_specs,
        out_shape=out_shape,
    )(*row_ins, *whole_ins)


def _matmul(a, w):
    return lax.dot_general(a, w, (((1,), (0,)), ((), ())),
                           preferred_element_type=jnp.float32)


def kernel(x, edge_index, batch,
           W1, b1, W2, b2, W3, b3, W4, b4, W5, b5, W6, b6,
           W7, b7, W8, b8, W9, b9, W10, b10, W11, b11, W12, b12):
    Ws = [W1, W2, W3, W4, W5, W6, W7, W8, W9, W10, W11, W12]
    bs = [b.reshape(1, -1) for b in
          [b1, b2, b3, b4, b5, b6, b7, b8, b9, b10, b11, b12]]

    sentinel = jnp.full((_EP - _E,), _N, jnp.int32)
    srcf = jnp.concatenate([edge_index[0], sentinel])
    dstf = jnp.concatenate([edge_index[1], sentinel])
    xp = jnp.pad(x, ((0, _NP - _N), (0, 0)))

    def agg(table, w):
        if _mega_depth(w) is not None:
            return _sc_aggregate_mega(table, srcf, dstf, w)
        return _sc_aggregate(table, srcf, dstf, w)

    P = agg(jnp.ones((_NP, 8), jnp.float32), 8)

    def s0(p0, p1, xb):
        dinv = lax.rsqrt(p0[:, :1] + p1[:, :1] + 1.0)
        return jnp.broadcast_to(dinv, (dinv.shape[0], 8)), dinv * xb

    dinv8, u = _rows_call(s0, [P[0], P[1], xp], [], [8, 8])

    for i in range(5):
        P = agg(u, u.shape[1])

        def sexp(p0, p1, ub, dv, W, b):
            dvc = dv[:, :1]
            h = jnp.tanh(_matmul(dvc * (p0 + p1 + ub), W) + b)
            return dvc * h

        u = _rows_call(sexp, [P[0], P[1], u, dinv8], [Ws[i], bs[i]],
                       [Ws[i].shape[1]])[0]

    u6L, u6R = u[:, :128], u[:, 128:]
    PL = agg(u6L, 128)
    PR = agg(u6R, 128)
    W7a, W7b = W7[:512], W7[512:]

    def s6(p0l, p1l, p0r, p1r, ul, ur, dv, xb, W6_, b6_, W7a_, W7b_):
        dvc = dv[:, :1]
        z = jnp.concatenate([dvc * (p0l + p1l + ul), dvc * (p0r + p1r + ur)],
                            axis=1)
        h6 = jnp.tanh(_matmul(z, W6_) + b6_)
        g7 = dvc * (_matmul(h6, W7a_) + _matmul(xb, W7b_))
        return g7[:, :128], g7[:, 128:]

    g7L, g7R = _rows_call(s6, [PL[0], PL[1], PR[0], PR[1], u6L, u6R, dinv8, xp],
                          [Ws[5], bs[5], W7a, W7b], [128, 128])

    PL = agg(g7L, 128)
    PR = agg(g7R, 128)

    def s7(p0l, p1l, p0r, p1r, gl, gr, dv, W8_, b7_):
        dvc = dv[:, :1]
        conv = jnp.concatenate([dvc * (p0l + p1l + gl),
                                dvc * (p0r + p1r + gr)], axis=1) + b7_
        return dvc * _matmul(jnp.tanh(conv), W8_)

    g = _rows_call(s7, [PL[0], PL[1], PR[0], PR[1], g7L, g7R, dinv8],
                   [Ws[7], bs[6]], [128])[0]

    for L in range(8, 12):
        P = agg(g, g.shape[1])

        def scon(p0, p1, gb, dv, Wn, bl):
            dvc = dv[:, :1]
            h = jnp.tanh(dvc * (p0 + p1 + gb) + bl)
            return dvc * _matmul(h, Wn)

        g = _rows_call(scon, [P[0], P[1], g, dinv8], [Ws[L], bs[L - 1]],
                       [Ws[L].shape[1]])[0]

    P = agg(g, 8)

    def sfin(p0, p1, gb, dv, bl):
        return dv[:, :1] * (p0 + p1 + gb) + bl

    out = _rows_call(sfin, [P[0], P[1], g, dinv8], [bs[11]], [8])[0]
    return out[:_N]


# speedup vs baseline: 1.0972x; 1.0972x over previous
import functools

import jax
import jax.numpy as jnp
from jax import lax
from jax.experimental import pallas as pl
from jax.experimental.pallas import tpu as pltpu
from jax.experimental.pallas import tpu_sc as plsc

_N = 10000
_NP = 10240
_E = 160000
_EP = 163840
_NC = 2
_NS = 16
_EPW = _EP // (_NC * _NS)
_C = 128
_RPT = _NP // _NS
_BR = 512


_CPW = _EPW // _C


def _mega_depth(w):
    if w <= 16:
        return _EPW
    if w == 32:
        return _EPW // 2
    if w == 64:
        return _EPW // 5
    return None


def _sc_aggregate_mega(table, srcf, dstf, w):
    D = _mega_depth(w)
    nd = _EPW // D
    mesh = plsc.VectorSubcoreMesh(core_axis_name="c", subcore_axis_name="s")

    @functools.partial(
        pl.kernel,
        out_type=jax.ShapeDtypeStruct((_NC, _NP, w), jnp.float32),
        mesh=mesh,
        scratch_types=[
            pltpu.VMEM((_EPW,), jnp.int32),
            pltpu.VMEM((_EPW,), jnp.int32),
            pltpu.VMEM((D, w), jnp.float32),
            pltpu.VMEM_SHARED((_NP, w), jnp.float32),
            pltpu.SemaphoreType.DMA,
            pltpu.SemaphoreType.DMA,
        ],
        compiler_params=pltpu.CompilerParams(use_tc_tiling_on_sc=False),
    )
    def agg(table_h, src_h, dst_h, zero_h, out_h, sidx, didx, rows, acc, g, s):
        cid = lax.axis_index("c")
        sid = lax.axis_index("s")
        wid = cid * _NS + sid

        ibase = pl.multiple_of(wid * _EPW, 8)
        pltpu.sync_copy(src_h.at[pl.ds(ibase, _EPW)], sidx)
        pltpu.sync_copy(dst_h.at[pl.ds(ibase, _EPW)], didx)

        pltpu.sync_copy(zero_h, rows.at[pl.ds(0, _C)])
        for k in range(_RPT // _C):
            off = pl.multiple_of(sid * _RPT + k * _C, _C)
            pltpu.sync_copy(rows.at[pl.ds(0, _C)], acc.at[pl.ds(off, _C)])
        plsc.subcore_barrier()

        if nd == 1:
            pltpu.async_copy(table_h.at[sidx], rows, g).wait()
            pltpu.async_copy(rows, acc.at[didx], s, add=True).wait()
        else:
            for d in range(nd):
                sl = pl.ds(d * D, D)
                pltpu.async_copy(table_h.at[sidx.at[sl]], rows, g).wait()
                pltpu.async_copy(rows, acc.at[didx.at[sl]], s, add=True).wait()
        plsc.subcore_barrier()

        nwb = _RPT // _C
        st = [rows.at[pl.ds(0, _C)], rows.at[pl.ds(_C, _C)]]
        sem = [g, s]
        for k in range(nwb):
            off = pl.multiple_of(sid * _RPT + k * _C, _C)
            b = k % 2
            if k >= 2:
                poff = pl.multiple_of(sid * _RPT + (k - 2) * _C, _C)
                pltpu.make_async_copy(st[b], out_h.at[cid, pl.ds(poff, _C)],
                                      sem[b]).wait()
            pltpu.async_copy(acc.at[pl.ds(off, _C)], st[b], sem[b]).wait()
            pltpu.async_copy(st[b], out_h.at[cid, pl.ds(off, _C)], sem[b])
        for k in range(nwb - 2, nwb):
            off = pl.multiple_of(sid * _RPT + k * _C, _C)
            pltpu.make_async_copy(st[k % 2], out_h.at[cid, pl.ds(off, _C)],
                                  sem[k % 2]).wait()

    return agg(table, srcf, dstf, jnp.zeros((_C, w), jnp.float32))


_NBR = 2
_NRR = _CPW // _NBR


def _sc_aggregate(table, srcp, dstp, w):
    mesh = plsc.VectorSubcoreMesh(core_axis_name="c", subcore_axis_name="s")

    @functools.partial(
        pl.kernel,
        out_type=jax.ShapeDtypeStruct((_NC, _NP, w), jnp.float32),
        mesh=mesh,
        scratch_types=[
            pltpu.VMEM((_CPW, _C), jnp.int32),
            pltpu.VMEM((_CPW, _C), jnp.int32),
            *([pltpu.VMEM((_C, w), jnp.float32)] * _NBR),
            pltpu.VMEM_SHARED((_NP, w), jnp.float32),
            *([pltpu.SemaphoreType.DMA] * (2 * _NBR)),
        ],
        compiler_params=pltpu.CompilerParams(use_tc_tiling_on_sc=False),
    )
    def agg(table_h, src_h, dst_h, zero_h, out_h,
            sidx, didx, r0, r1, acc, g0, g1, s0, s1):
        rows = [r0, r1]
        gsem = [g0, g1]
        ssem = [s0, s1]
        cid = lax.axis_index("c")
        sid = lax.axis_index("s")
        wid = cid * _NS + sid

        ibase = pl.multiple_of(wid * _CPW, 8)
        pltpu.sync_copy(src_h.at[pl.ds(ibase, _CPW)], sidx)
        pltpu.sync_copy(dst_h.at[pl.ds(ibase, _CPW)], didx)

        pltpu.sync_copy(zero_h, rows[0])
        for k in range(_RPT // _C):
            off = pl.multiple_of(sid * _RPT + k * _C, _C)
            pltpu.sync_copy(rows[0], acc.at[pl.ds(off, _C)])
        for b in range(_NBR):
            pltpu.async_copy(table_h.at[sidx.at[b]], rows[b], gsem[b])
        plsc.subcore_barrier()

        def rnd(r, carry):
            for b in range(_NBR):
                k = r * _NBR + b
                pltpu.make_async_copy(table_h.at[sidx.at[k]], rows[b],
                                      gsem[b]).wait()
                pltpu.async_copy(rows[b], acc.at[didx.at[k]], ssem[b],
                                 add=True)
            for b in range(_NBR):
                k = (r + 1) * _NBR + b
                pltpu.make_async_copy(rows[b], acc.at[didx.at[k - _NBR]],
                                      ssem[b]).wait()
                pltpu.async_copy(table_h.at[sidx.at[k]], rows[b], gsem[b])
            return carry

        lax.fori_loop(0, _NRR - 1, rnd, 0)
        for b in range(_NBR):
            k = (_NRR - 1) * _NBR + b
            pltpu.make_async_copy(table_h.at[sidx.at[k]], rows[b],
                                  gsem[b]).wait()
            pltpu.async_copy(rows[b], acc.at[didx.at[k]], ssem[b], add=True)
        for b in range(_NBR):
            k = (_NRR - 1) * _NBR + b
            pltpu.make_async_copy(rows[b], acc.at[didx.at[k]],
                                  ssem[b]).wait()
        plsc.subcore_barrier()

        nwb = _RPT // _C
        for k in range(nwb):
            off = pl.multiple_of(sid * _RPT + k * _C, _C)
            b = k % _NBR
            if k >= _NBR:
                poff = pl.multiple_of(sid * _RPT + (k - _NBR) * _C, _C)
                pltpu.make_async_copy(rows[b], out_h.at[cid, pl.ds(poff, _C)],
                                      ssem[b]).wait()
            pltpu.async_copy(acc.at[pl.ds(off, _C)], rows[b], gsem[b]).wait()
            pltpu.async_copy(rows[b], out_h.at[cid, pl.ds(off, _C)], ssem[b])
        for k in range(nwb - _NBR, nwb):
            off = pl.multiple_of(sid * _RPT + k * _C, _C)
            pltpu.make_async_copy(rows[k % _NBR], out_h.at[cid, pl.ds(off, _C)],
                                  ssem[k % _NBR]).wait()

    return agg(table, srcp, dstp, jnp.zeros((_C, w), jnp.float32))


_NPC = _EP // _NS
_CPP = _NPC // _C


def _sc_aggregate_pair(tableLR, srcpL, srcpH, dstp):
    w = 128
    mesh = plsc.VectorSubcoreMesh(core_axis_name="c", subcore_axis_name="s")

    @functools.partial(
        pl.kernel,
        out_type=jax.ShapeDtypeStruct((_NC, _NP, w), jnp.float32),
        mesh=mesh,
        scratch_types=[
            pltpu.VMEM((_CPP, _C), jnp.int32),
            pltpu.VMEM((2, _C), jnp.int32),
            pltpu.VMEM((2, _C), jnp.int32),
            *([pltpu.VMEM((_C, w), jnp.float32)] * 2),
            pltpu.VMEM_SHARED((_NP, w), jnp.float32),
            *([pltpu.SemaphoreType.DMA] * 6),
        ],
        compiler_params=pltpu.CompilerParams(use_tc_tiling_on_sc=False),
    )
    def agg(tab_h, srcL_h, srcH_h, dst_h, zero_h, out_h,
            sidx, db0, db1, r0, r1, acc, g0, g1, s0, s1, d0, d1):
        rows = [r0, r1]
        gsem = [g0, g1]
        ssem = [s0, s1]
        dbuf = [db0, db1]
        dsem = [d0, d1]
        cid = lax.axis_index("c")
        sid = lax.axis_index("s")

        ibase = pl.multiple_of(sid * _CPP, 8)

        @pl.when(cid == 0)
        def _():
            pltpu.sync_copy(srcL_h.at[pl.ds(ibase, _CPP)], sidx)

        @pl.when(cid == 1)
        def _():
            pltpu.sync_copy(srcH_h.at[pl.ds(ibase, _CPP)], sidx)

        for p in range(2):
            pltpu.async_copy(dst_h.at[pl.ds(ibase + 2 * p, 2)], dbuf[p],
                             dsem[p])
        pltpu.sync_copy(zero_h, rows[0])
        for k in range(_RPT // _C):
            off = pl.multiple_of(sid * _RPT + k * _C, _C)
            pltpu.sync_copy(rows[0], acc.at[pl.ds(off, _C)])
        for b in range(2):
            pltpu.async_copy(tab_h.at[sidx.at[b]], rows[b], gsem[b])
        plsc.subcore_barrier()

        def halfround(t, p, issue_gather, prefetch):
            base = 4 * t + 2 * p
            pltpu.make_async_copy(dst_h.at[pl.ds(ibase + base, 2)], dbuf[p],
                                  dsem[p]).wait()
            for b in range(2):
                k = base + b
                pltpu.make_async_copy(tab_h.at[sidx.at[k]], rows[b],
                                      gsem[b]).wait()
                pltpu.async_copy(rows[b], acc.at[dbuf[p].at[b]], ssem[b],
                                 add=True)
            for b in range(2):
                k = base + 2 + b
                pltpu.make_async_copy(rows[b], acc.at[dbuf[p].at[b]],
                                      ssem[b]).wait()
                if issue_gather:
                    pltpu.async_copy(tab_h.at[sidx.at[k]], rows[b], gsem[b])
            if prefetch:
                pltpu.async_copy(dst_h.at[pl.ds(ibase + base + 4, 2)],
                                 dbuf[p], dsem[p])

        def pairrnd(t, carry):
            halfround(t, 0, True, True)
            halfround(t, 1, True, True)
            return carry

        lax.fori_loop(0, _CPP // 4 - 1, pairrnd, 0)
        t_last = _CPP // 4 - 1
        halfround(t_last, 0, True, False)
        halfround(t_last, 1, False, False)
        plsc.subcore_barrier()

        nwb = _RPT // _C
        for k in range(nwb):
            off = pl.multiple_of(sid * _RPT + k * _C, _C)
            b = k % 2
            if k >= 2:
                poff = pl.multiple_of(sid * _RPT + (k - 2) * _C, _C)
                pltpu.make_async_copy(rows[b], out_h.at[cid, pl.ds(poff, _C)],
                                      ssem[b]).wait()
            pltpu.async_copy(acc.at[pl.ds(off, _C)], rows[b], gsem[b]).wait()
            pltpu.async_copy(rows[b], out_h.at[cid, pl.ds(off, _C)], ssem[b])
        for k in range(nwb - 2, nwb):
            off = pl.multiple_of(sid * _RPT + k * _C, _C)
            pltpu.make_async_copy(rows[k % 2], out_h.at[cid, pl.ds(off, _C)],
                                  ssem[k % 2]).wait()

    return agg(tableLR, srcpL, srcpH, dstp, jnp.zeros((_C, 128), jnp.float32))


def _rows_call(fn, row_ins, whole_ins, out_widths):
    nb = _NP // _BR
    nri, nwi = len(row_ins), len(whole_ins)
    in_specs = (
        [pl.BlockSpec((_BR, a.shape[1]), lambda i: (i, 0)) for a in row_ins]
        + [pl.BlockSpec(a.shape, lambda i, _nd=a.ndim: (0,) * _nd) for a in whole_ins]
    )
    out_specs = [pl.BlockSpec((_BR, w), lambda i: (i, 0)) for w in out_widths]
    out_shape = [jax.ShapeDtypeStruct((_NP, w), jnp.float32) for w in out_widths]

    def body(*refs):
        ins = [r[...] for r in refs[: nri + nwi]]
        outs = fn(*ins)
        if not isinstance(outs, (tuple, list)):
            outs = (outs,)
        for r, o in zip(refs[nri + nwi:], outs):
            r[...] = o

    return pl.pallas_call(
        body,
        grid=(nb,),
        in_specs=in_specs,
        out_specs=out_specs,
        out_shape=out_shape,
    )(*row_ins, *whole_ins)


def _matmul(a, w):
    return lax.dot_general(a, w, (((1,), (0,)), ((), ())),
                           preferred_element_type=jnp.float32)


def kernel(x, edge_index, batch,
           W1, b1, W2, b2, W3, b3, W4, b4, W5, b5, W6, b6,
           W7, b7, W8, b8, W9, b9, W10, b10, W11, b11, W12, b12):
    Ws = [W1, W2, W3, W4, W5, W6, W7, W8, W9, W10, W11, W12]
    bs = [b.reshape(1, -1) for b in
          [b1, b2, b3, b4, b5, b6, b7, b8, b9, b10, b11, b12]]

    sentinel = jnp.full((_EP - _E,), _N, jnp.int32)
    srcf = jnp.concatenate([edge_index[0], sentinel])
    dstf = jnp.concatenate([edge_index[1], sentinel])
    xp = jnp.pad(x, ((0, _NP - _N), (0, 0)))

    srcp = srcf.reshape(_EP // _C, _C)
    dstp = dstf.reshape(_EP // _C, _C)
    srcpH = (srcf + _NP).reshape(_EP // _C, _C)

    def agg(table, w):
        if _mega_depth(w) is not None:
            return _sc_aggregate_mega(table, srcf, dstf, w)
        return _sc_aggregate(table, srcp, dstp, w)

    P = agg(jnp.ones((_NP, 8), jnp.float32), 8)

    def s0(p0, p1, xb):
        dinv = lax.rsqrt(p0[:, :1] + p1[:, :1] + 1.0)
        return jnp.broadcast_to(dinv, (dinv.shape[0], 8)), dinv * xb

    dinv8, u = _rows_call(s0, [P[0], P[1], xp], [], [8, 8])

    for i in range(5):
        P = agg(u, u.shape[1])

        def sexp(p0, p1, ub, dv, W, b):
            dvc = dv[:, :1]
            h = jnp.tanh(_matmul(dvc * (p0 + p1 + ub), W) + b)
            return dvc * h

        u = _rows_call(sexp, [P[0], P[1], u, dinv8], [Ws[i], bs[i]],
                       [Ws[i].shape[1]])[0]

    u6L, u6R = u[:, :128], u[:, 128:]
    P6 = _sc_aggregate_pair(jnp.concatenate([u6L, u6R], axis=0),
                            srcp, srcpH, dstp)
    W7a, W7b = W7[:512], W7[512:]

    def s6(pl_, pr_, ul, ur, dv, xb, W6_, b6_, W7a_, W7b_):
        dvc = dv[:, :1]
        z = jnp.concatenate([dvc * (pl_ + ul), dvc * (pr_ + ur)], axis=1)
        h6 = jnp.tanh(_matmul(z, W6_) + b6_)
        g7 = dvc * (_matmul(h6, W7a_) + _matmul(xb, W7b_))
        return g7[:, :128], g7[:, 128:]

    g7L, g7R = _rows_call(s6, [P6[0], P6[1], u6L, u6R, dinv8, xp],
                          [Ws[5], bs[5], W7a, W7b], [128, 128])

    P7 = _sc_aggregate_pair(jnp.concatenate([g7L, g7R], axis=0),
                            srcp, srcpH, dstp)

    def s7(pl_, pr_, gl, gr, dv, W8_, b7_):
        dvc = dv[:, :1]
        conv = jnp.concatenate([dvc * (pl_ + gl),
                                dvc * (pr_ + gr)], axis=1) + b7_
        return dvc * _matmul(jnp.tanh(conv), W8_)

    g = _rows_call(s7, [P7[0], P7[1], g7L, g7R, dinv8],
                   [Ws[7], bs[6]], [128])[0]

    for L in range(8, 12):
        P = agg(g, g.shape[1])

        def scon(p0, p1, gb, dv, Wn, bl):
            dvc = dv[:, :1]
            h = jnp.tanh(dvc * (p0 + p1 + gb) + bl)
            return dvc * _matmul(h, Wn)

        g = _rows_call(scon, [P[0], P[1], g, dinv8], [Ws[L], bs[L - 1]],
                       [Ws[L].shape[1]])[0]

    P = agg(g, 8)

    def sfin(p0, p1, gb, dv, bl):
        return dv[:, :1] * (p0 + p1 + gb) + bl

    out = _rows_call(sfin, [P[0], P[1], g, dinv8], [bs[11]], [8])[0]
    return out[:_N]
